# baseline (speedup 1.0000x reference)
import dataclasses
import functools

import jax
import jax.numpy as jnp
from jax import lax
from jax.experimental import pallas as pl
from jax.experimental.pallas import tpu as pltpu
from jax.experimental.pallas import tpu_sc as plsc

NC = 2
NS = 16
NW = NC * NS
L = 16
CH = 128
DEG_D = 128


def _vector_mesh():
    return plsc.VectorSubcoreMesh(core_axis_name="c", subcore_axis_name="s")


def _sc_compiler_params():
    cp = pltpu.CompilerParams()
    if "needs_layout_passes" in pltpu.CompilerParams.__dataclass_fields__:
        cp = dataclasses.replace(cp, needs_layout_passes=False)
    return cp


def _row_chunks(n, sid, do_copy):
    n_full = n // CH
    rem = n % CH
    n_tot = n_full + (1 if rem else 0)
    nt = -(-n_tot // NS)

    @pl.loop(0, nt)
    def _(t):
        q = t * NS + sid

        @pl.when(q < n_full)
        def _():
            do_copy(q * CH, CH)

        if rem:
            @pl.when(q == n_full)
            def _():
                do_copy(n_full * CH, rem)


def _sc_deg_partial(col, w, n, d):
    e = col.shape[0]
    n_chunks = e // CH
    n_loc = n_chunks // NW

    @functools.partial(
        pl.kernel,
        mesh=_vector_mesh(),
        out_type=jax.ShapeDtypeStruct((NC, n, d), jnp.float32),
        compiler_params=_sc_compiler_params(),
        scratch_types=[
            pltpu.VMEM((CH,), jnp.int32),
            pltpu.VMEM((CH,), jnp.float32),
            pltpu.VMEM((CH, d), jnp.float32),
            pltpu.VMEM_SHARED((n, d), jnp.float32),
        ],
    )
    def k(col_hbm, w_hbm, out_hbm, cidx, wv, rbuf, acc):
        cid = lax.axis_index("c")
        sid = lax.axis_index("s")
        wid = sid * NC + cid
        zero = jnp.zeros((L,), jnp.float32)

        @pl.loop(0, CH)
        def _(r):
            for j in range(d // L):
                rbuf[r, pl.ds(j * L, L)] = zero

        def _zero(off, size):
            off = pl.multiple_of(off, 8)
            pltpu.sync_copy(rbuf.at[pl.ds(0, size)], acc.at[pl.ds(off, size)])

        _row_chunks(n, sid, _zero)
        plsc.subcore_barrier()

        @pl.loop(0, n_loc)
        def _(t):
            base = (t * NW + wid) * CH
            pltpu.sync_copy(col_hbm.at[pl.ds(base, CH)], cidx)
            pltpu.sync_copy(w_hbm.at[pl.ds(base, CH)], wv)

            @pl.loop(0, CH)
            def _(r):
                s = plsc.load_gather(wv, [jnp.full((L,), r, jnp.int32)])
                for j in range(d // L):
                    rbuf[r, pl.ds(j * L, L)] = s

            pltpu.sync_copy(rbuf, acc.at[cidx], add=True)

        plsc.subcore_barrier()

        def _wb(off, size):
            off = pl.multiple_of(off, 8)
            pltpu.sync_copy(acc.at[pl.ds(off, size)],
                            out_hbm.at[cid, pl.ds(off, size)])

        _row_chunks(n, sid, _wb)

    return k(col, w)


TPC = 1
TE = TPC * CH
EDGE_PAD = 2 * TE * NW


def _sc_agg(g, row, col, w, n):
    e = row.shape[0]
    d = g.shape[1]
    nt = e // (TE * NW)

    row_bytes = d * 4
    chunk_bytes = CH * row_bytes

    @functools.partial(
        pl.kernel,
        mesh=_vector_mesh(),
        out_type=jax.ShapeDtypeStruct((NC, n, d), jnp.float32),
        compiler_params=_sc_compiler_params(),
        scratch_types=(
            [pltpu.VMEM((CH,), jnp.int32)] * (2 * TPC * 2)
            + [pltpu.VMEM((TE,), jnp.float32)] * 2
            + [pltpu.VMEM((CH, d), jnp.float32)] * (TPC * 2)
            + [pltpu.VMEM_SHARED((n, d), jnp.float32)]
            + [pltpu.SemaphoreType.DMA] * 4
        ),
    )
    def k(g_hbm, row_hbm, col_hbm, w_hbm, out_hbm, *sc):
        ridx = [sc[0:TPC], sc[TPC:2 * TPC]]
        cidx = [sc[2 * TPC:3 * TPC], sc[3 * TPC:4 * TPC]]
        wv = [sc[4 * TPC], sc[4 * TPC + 1]]
        rb = [sc[4 * TPC + 2:4 * TPC + 2 + TPC],
              sc[4 * TPC + 2 + TPC:4 * TPC + 2 + 2 * TPC]]
        acc = sc[4 * TPC + 2 + 2 * TPC]
        gsem = [sc[-4], sc[-3]]
        ssem = [sc[-2], sc[-1]]

        cid = lax.axis_index("c")
        sid = lax.axis_index("s")
        wid = sid * NC + cid
        zero = jnp.zeros((L,), jnp.float32)

        @pl.loop(0, CH)
        def _(r):
            for j in range(d // L):
                rb[0][0][r, pl.ds(j * L, L)] = zero

        def _zero(off, size):
            off = pl.multiple_of(off, 8)
            pltpu.sync_copy(rb[0][0].at[pl.ds(0, size)], acc.at[pl.ds(off, size)])

        _row_chunks(n, sid, _zero)
        plsc.subcore_barrier()

        def prep(u, s):
            base = u * NW + wid
            for c in range(TPC):
                off = pl.multiple_of((base * TPC + c) * CH, 8)
                pltpu.sync_copy(row_hbm.at[pl.ds(off, CH)], ridx[s][c])
                pltpu.sync_copy(col_hbm.at[pl.ds(off, CH)], cidx[s][c])
            woff = pl.multiple_of(base * TE, 8)
            pltpu.sync_copy(w_hbm.at[pl.ds(woff, TE)], wv[s])
            for c in range(TPC):
                pltpu.async_copy(g_hbm.at[ridx[s][c]], rb[s][c], gsem[s])

        def wait_scatter(s):
            del s

        def finish(s):
            for c in range(TPC):
                pltpu.make_async_copy(g_hbm.at[ridx[s][c]], rb[s][c],
                                      gsem[s]).wait()
            for c in range(TPC):
                buf = rb[s][c]

                @pl.loop(0, CH, step=2)
                def _(r):
                    for rr in range(2):
                        sv = plsc.load_gather(
                            wv[s], [jnp.full((L,), r + rr + c * CH, jnp.int32)])
                        for j in range(d // L):
                            buf[r + rr, pl.ds(j * L, L)] = (
                                buf[r + rr, pl.ds(j * L, L)] * sv)

            for c in range(TPC):
                pltpu.sync_copy(rb[s][c], acc.at[cidx[s][c]], add=True)

        prep(0, 0)
        prep(1, 1)

        @pl.loop(0, nt // 2)
        def _(k_it):
            u0 = k_it * 2

            finish(0)

            @pl.when(u0 + 2 < nt)
            def _():
                wait_scatter(0)
                prep(u0 + 2, 0)

            finish(1)

            @pl.when(u0 + 3 < nt)
            def _():
                wait_scatter(1)
                prep(u0 + 3, 1)

        wait_scatter(0)
        wait_scatter(1)
        plsc.subcore_barrier()

        def _wb(off, size):
            off = pl.multiple_of(off, 8)
            pltpu.sync_copy(acc.at[pl.ds(off, size)],
                            out_hbm.at[cid, pl.ds(off, size)])

        _row_chunks(n, sid, _wb)

    return k(g, row, col, w)


_R = 1000


def _tc_prep(x, w1, degp):
    n, d = x.shape

    def body(x_ref, w_ref, dp_ref, g_ref, dis_ref):
        d0 = dp_ref[0]
        d1 = dp_ref[1]
        nl = d0.shape[1]
        deg = (jnp.sum(d0, axis=1) + jnp.sum(d1, axis=1)) * (1.0 / nl) + 1.0
        good = deg > 0.0
        dis = jnp.where(good, lax.rsqrt(jnp.where(good, deg, 1.0)), 0.0)
        dis2 = jnp.broadcast_to(dis[:, None], (_R, d))
        h = jnp.dot(x_ref[...], w_ref[...], preferred_element_type=jnp.float32)
        g_ref[...] = h * dis2
        dis_ref[...] = dis2

    return pl.pallas_call(
        body,
        grid=(n // _R,),
        in_specs=[
            pl.BlockSpec((_R, d), lambda i: (i, 0)),
            pl.BlockSpec((d, d), lambda i: (0, 0)),
            pl.BlockSpec((2, _R, DEG_D), lambda i: (0, i, 0)),
        ],
        out_specs=[
            pl.BlockSpec((_R, d), lambda i: (i, 0)),
            pl.BlockSpec((_R, d), lambda i: (i, 0)),
        ],
        out_shape=[
            jax.ShapeDtypeStruct((n, d), jnp.float32),
            jax.ShapeDtypeStruct((n, d), jnp.float32),
        ],
    )(x, w1, degp)


def _tc_mid(agg, g, dis, b, w_next):
    n, d = g.shape

    def body(a_ref, g_ref, dis_ref, b_ref, w_ref, o_ref):
        s = a_ref[0] + a_ref[1] + g_ref[...]
        pre = s * dis_ref[...] + b_ref[...]
        t = jnp.maximum(pre, 0.01 * pre)
        o_ref[...] = jnp.dot(
            t, w_ref[...], preferred_element_type=jnp.float32) * dis_ref[...]

    return pl.pallas_call(
        body,
        grid=(n // _R,),
        in_specs=[
            pl.BlockSpec((2, _R, d), lambda i: (0, i, 0)),
            pl.BlockSpec((_R, d), lambda i: (i, 0)),
            pl.BlockSpec((_R, d), lambda i: (i, 0)),
            pl.BlockSpec((1, d), lambda i: (0, 0)),
            pl.BlockSpec((d, d), lambda i: (0, 0)),
        ],
        out_specs=pl.BlockSpec((_R, d), lambda i: (i, 0)),
        out_shape=jax.ShapeDtypeStruct((n, d), jnp.float32),
    )(agg, g, dis, b, w_next)


def _tc_final(agg, g, dis, b):
    n, d = g.shape

    def body(a_ref, g_ref, dis_ref, b_ref, o_ref):
        s = a_ref[0] + a_ref[1] + g_ref[...]
        o_ref[...] = s * dis_ref[...] + b_ref[...]

    return pl.pallas_call(
        body,
        grid=(n // _R,),
        in_specs=[
            pl.BlockSpec((2, _R, d), lambda i: (0, i, 0)),
            pl.BlockSpec((_R, d), lambda i: (i, 0)),
            pl.BlockSpec((_R, d), lambda i: (i, 0)),
            pl.BlockSpec((1, d), lambda i: (0, 0)),
        ],
        out_specs=pl.BlockSpec((_R, d), lambda i: (i, 0)),
        out_shape=jax.ShapeDtypeStruct((n, d), jnp.float32),
    )(agg, g, dis, b)


def kernel(x, edge_index, edge_weight, W1, b1, W2, b2, W3, b3, W4, b4):
    n, d = x.shape
    row = edge_index[0]
    col = edge_index[1]
    w = edge_weight.astype(jnp.float32)

    e = row.shape[0]
    pad = (-e) % EDGE_PAD
    if pad:
        zi = jnp.zeros((pad,), row.dtype)
        row = jnp.concatenate([row, zi])
        col = jnp.concatenate([col, zi])
        w = jnp.concatenate([w, jnp.zeros((pad,), w.dtype)])

    degp = _sc_deg_partial(col, w, n, DEG_D)
    g, dis = _tc_prep(x, W1, degp)
    agg = _sc_agg(g, row, col, w, n)
    g = _tc_mid(agg, g, dis, b1.reshape(1, d), W2)
    agg = _sc_agg(g, row, col, w, n)
    g = _tc_mid(agg, g, dis, b2.reshape(1, d), W3)
    agg = _sc_agg(g, row, col, w, n)
    g = _tc_mid(agg, g, dis, b3.reshape(1, d), W4)
    agg = _sc_agg(g, row, col, w, n)
    return _tc_final(agg, g, dis, b4.reshape(1, d))


# speedup vs baseline: 1.1987x; 1.1987x over previous
import dataclasses
import functools

import jax
import jax.numpy as jnp
from jax import lax
from jax.experimental import pallas as pl
from jax.experimental.pallas import tpu as pltpu
from jax.experimental.pallas import tpu_sc as plsc

NC = 2
NS = 16
NW = NC * NS
L = 16
CH = 128
DEG_D = 128


def _vector_mesh():
    return plsc.VectorSubcoreMesh(core_axis_name="c", subcore_axis_name="s")


def _sc_compiler_params():
    cp = pltpu.CompilerParams()
    if "needs_layout_passes" in pltpu.CompilerParams.__dataclass_fields__:
        cp = dataclasses.replace(cp, needs_layout_passes=False)
    return cp


def _row_chunks(n, sid, do_copy):
    n_full = n // CH
    rem = n % CH
    n_tot = n_full + (1 if rem else 0)
    nt = -(-n_tot // NS)

    @pl.loop(0, nt)
    def _(t):
        q = t * NS + sid

        @pl.when(q < n_full)
        def _():
            do_copy(q * CH, CH)

        if rem:
            @pl.when(q == n_full)
            def _():
                do_copy(n_full * CH, rem)


def _sc_deg_partial(col, w, n, d):
    e = col.shape[0]
    n_chunks = e // CH
    n_loc = n_chunks // NW

    @functools.partial(
        pl.kernel,
        mesh=_vector_mesh(),
        out_type=jax.ShapeDtypeStruct((NC, n, d), jnp.float32),
        compiler_params=_sc_compiler_params(),
        scratch_types=[
            pltpu.VMEM((CH,), jnp.int32),
            pltpu.VMEM((CH,), jnp.float32),
            pltpu.VMEM((CH, d), jnp.float32),
            pltpu.VMEM_SHARED((n, d), jnp.float32),
        ],
    )
    def k(col_hbm, w_hbm, out_hbm, cidx, wv, rbuf, acc):
        cid = lax.axis_index("c")
        sid = lax.axis_index("s")
        wid = sid * NC + cid
        zero = jnp.zeros((L,), jnp.float32)

        @pl.loop(0, CH)
        def _(r):
            for j in range(d // L):
                rbuf[r, pl.ds(j * L, L)] = zero

        def _zero(off, size):
            off = pl.multiple_of(off, 8)
            pltpu.sync_copy(rbuf.at[pl.ds(0, size)], acc.at[pl.ds(off, size)])

        _row_chunks(n, sid, _zero)
        plsc.subcore_barrier()

        @pl.loop(0, n_loc)
        def _(t):
            base = (t * NW + wid) * CH
            pltpu.sync_copy(col_hbm.at[pl.ds(base, CH)], cidx)
            pltpu.sync_copy(w_hbm.at[pl.ds(base, CH)], wv)

            @pl.loop(0, CH)
            def _(r):
                s = plsc.load_gather(wv, [jnp.full((L,), r, jnp.int32)])
                for j in range(d // L):
                    rbuf[r, pl.ds(j * L, L)] = s

            pltpu.sync_copy(rbuf, acc.at[cidx], add=True)

        plsc.subcore_barrier()

        def _wb(off, size):
            off = pl.multiple_of(off, 8)
            pltpu.sync_copy(acc.at[pl.ds(off, size)],
                            out_hbm.at[cid, pl.ds(off, size)])

        _row_chunks(n, sid, _wb)

    return k(col, w)


CPB = 8
BE = CPB * CH
EDGE_PAD = 2 * BE * NW


def _sc_agg(g, row2d, col2d, w2d, n):
    n_chunks = row2d.shape[0]
    d = g.shape[1]
    nb = n_chunks // (CPB * NW)

    @functools.partial(
        pl.kernel,
        mesh=_vector_mesh(),
        out_type=jax.ShapeDtypeStruct((NC, n, d), jnp.float32),
        compiler_params=_sc_compiler_params(),
        scratch_types=(
            [pltpu.VMEM((CPB, CH), jnp.int32)] * 4
            + [pltpu.VMEM((CPB, CH), jnp.float32)] * 2
            + [pltpu.VMEM((CH, d), jnp.float32)] * 2
            + [pltpu.VMEM_SHARED((n, d), jnp.float32)]
            + [pltpu.SemaphoreType.DMA] * 6
        ),
    )
    def k(g_hbm, row_hbm, col_hbm, w_hbm, out_hbm, *sc):
        ridx = sc[0:2]
        cidx = sc[2:4]
        wv = sc[4:6]
        rb = sc[6:8]
        acc = sc[8]
        isem = sc[9:11]
        gsem = sc[11:13]
        ssem = sc[13:15]

        cid = lax.axis_index("c")
        sid = lax.axis_index("s")
        wid = sid * NC + cid
        zero = jnp.zeros((L,), jnp.float32)

        @pl.loop(0, CH)
        def _(r):
            for j in range(d // L):
                rb[0][r, pl.ds(j * L, L)] = zero

        def _zero(off, size):
            off = pl.multiple_of(off, 8)
            pltpu.sync_copy(rb[0].at[pl.ds(0, size)], acc.at[pl.ds(off, size)])

        _row_chunks(n, sid, _zero)
        plsc.subcore_barrier()

        def issue_idx(t, s):
            cb = pl.multiple_of((t * NW + wid) * CPB, 8)
            pltpu.async_copy(row_hbm.at[pl.ds(cb, CPB)], ridx[s], isem[s])
            pltpu.async_copy(col_hbm.at[pl.ds(cb, CPB)], cidx[s], isem[s])
            pltpu.async_copy(w_hbm.at[pl.ds(cb, CPB)], wv[s], isem[s])

        def wait_idx(t, s):
            cb = pl.multiple_of((t * NW + wid) * CPB, 8)
            pltpu.make_async_copy(row_hbm.at[pl.ds(cb, CPB)], ridx[s],
                                  isem[s]).wait()
            pltpu.make_async_copy(col_hbm.at[pl.ds(cb, CPB)], cidx[s],
                                  isem[s]).wait()
            pltpu.make_async_copy(w_hbm.at[pl.ds(cb, CPB)], wv[s],
                                  isem[s]).wait()

        def issue_gather(s, bs, j):
            return pltpu.async_copy(g_hbm.at[ridx[bs].at[j]], rb[s], gsem[s])

        def scale(s, bs, j):
            buf = rb[s]

            @pl.loop(0, CH, step=2)
            def _(r):
                for rr in range(2):
                    sv = plsc.load_gather(
                        wv[bs], [jnp.full((L,), j, jnp.int32),
                                 jnp.full((L,), r + rr, jnp.int32)])
                    for jj in range(d // L):
                        buf[r + rr, pl.ds(jj * L, L)] = (
                            buf[r + rr, pl.ds(jj * L, L)] * sv)

        def issue_scatter(s, bs, j):
            return pltpu.async_copy(rb[s], acc.at[cidx[bs].at[j]], ssem[s],
                                    add=True)

        issue_idx(0, 0)
        issue_idx(1, 1)
        wait_idx(0, 0)
        issue_gather(0, 0, 0)

        def wait_g(s, bs, j):
            pltpu.make_async_copy(g_hbm.at[ridx[bs].at[j]], rb[s],
                                  gsem[s]).wait()

        def body2(t, bs, prefetch_idx, prefetch_gather, scat):
            for j in range(CPB):
                s = j % 2
                wait_g(s, bs, j)
                s_next = (j + 1) % 2
                last = j + 1 < CPB or prefetch_gather
                if scat[s_next] is not None and last:
                    scat[s_next].wait()
                    scat[s_next] = None
                if j + 1 < CPB:
                    issue_gather(s_next, bs, j + 1)
                elif prefetch_gather:
                    wait_idx(t + 1, 1 - bs)
                    issue_gather(s_next, 1 - bs, 0)
                scale(s, bs, j)
                scat[s] = issue_scatter(s, bs, j)
            if prefetch_idx:
                for s2 in range(2):
                    if scat[s2] is not None:
                        scat[s2].wait()
                        scat[s2] = None
                issue_idx(t + 2, bs)
            return scat

        if nb > 2:
            @pl.loop(0, nb // 2 - 1)
            def _(kk):
                t0 = kk * 2
                sc0 = body2(t0, 0, True, True, [None, None])
                body2(t0 + 1, 1, True, True, sc0)

        sc0 = body2(nb - 2, 0, False, True, [None, None])
        sc1 = body2(nb - 1, 1, False, False, sc0)
        for h in sc1:
            if h is not None:
                h.wait()

        plsc.subcore_barrier()

        def _wb(off, size):
            off = pl.multiple_of(off, 8)
            pltpu.sync_copy(acc.at[pl.ds(off, size)],
                            out_hbm.at[cid, pl.ds(off, size)])

        _row_chunks(n, sid, _wb)

    return k(g, row2d, col2d, w2d)


_R = 1000


def _tc_prep(x, w1, degp):
    n, d = x.shape

    def body(x_ref, w_ref, dp_ref, g_ref, dis_ref):
        d0 = dp_ref[0]
        d1 = dp_ref[1]
        nl = d0.shape[1]
        deg = (jnp.sum(d0, axis=1) + jnp.sum(d1, axis=1)) * (1.0 / nl) + 1.0
        good = deg > 0.0
        dis = jnp.where(good, lax.rsqrt(jnp.where(good, deg, 1.0)), 0.0)
        dis2 = jnp.broadcast_to(dis[:, None], (_R, d))
        h = jnp.dot(x_ref[...], w_ref[...], preferred_element_type=jnp.float32)
        g_ref[...] = h * dis2
        dis_ref[...] = dis2

    return pl.pallas_call(
        body,
        grid=(n // _R,),
        in_specs=[
            pl.BlockSpec((_R, d), lambda i: (i, 0)),
            pl.BlockSpec((d, d), lambda i: (0, 0)),
            pl.BlockSpec((2, _R, DEG_D), lambda i: (0, i, 0)),
        ],
        out_specs=[
            pl.BlockSpec((_R, d), lambda i: (i, 0)),
            pl.BlockSpec((_R, d), lambda i: (i, 0)),
        ],
        out_shape=[
            jax.ShapeDtypeStruct((n, d), jnp.float32),
            jax.ShapeDtypeStruct((n, d), jnp.float32),
        ],
    )(x, w1, degp)


def _tc_mid(agg, g, dis, b, w_next):
    n, d = g.shape

    def body(a_ref, g_ref, dis_ref, b_ref, w_ref, o_ref):
        s = a_ref[0] + a_ref[1] + g_ref[...]
        pre = s * dis_ref[...] + b_ref[...]
        t = jnp.maximum(pre, 0.01 * pre)
        o_ref[...] = jnp.dot(
            t, w_ref[...], preferred_element_type=jnp.float32) * dis_ref[...]

    return pl.pallas_call(
        body,
        grid=(n // _R,),
        in_specs=[
            pl.BlockSpec((2, _R, d), lambda i: (0, i, 0)),
            pl.BlockSpec((_R, d), lambda i: (i, 0)),
            pl.BlockSpec((_R, d), lambda i: (i, 0)),
            pl.BlockSpec((1, d), lambda i: (0, 0)),
            pl.BlockSpec((d, d), lambda i: (0, 0)),
        ],
        out_specs=pl.BlockSpec((_R, d), lambda i: (i, 0)),
        out_shape=jax.ShapeDtypeStruct((n, d), jnp.float32),
    )(agg, g, dis, b, w_next)


def _tc_final(agg, g, dis, b):
    n, d = g.shape

    def body(a_ref, g_ref, dis_ref, b_ref, o_ref):
        s = a_ref[0] + a_ref[1] + g_ref[...]
        o_ref[...] = s * dis_ref[...] + b_ref[...]

    return pl.pallas_call(
        body,
        grid=(n // _R,),
        in_specs=[
            pl.BlockSpec((2, _R, d), lambda i: (0, i, 0)),
            pl.BlockSpec((_R, d), lambda i: (i, 0)),
            pl.BlockSpec((_R, d), lambda i: (i, 0)),
            pl.BlockSpec((1, d), lambda i: (0, 0)),
        ],
        out_specs=pl.BlockSpec((_R, d), lambda i: (i, 0)),
        out_shape=jax.ShapeDtypeStruct((n, d), jnp.float32),
    )(agg, g, dis, b)


def kernel(x, edge_index, edge_weight, W1, b1, W2, b2, W3, b3, W4, b4):
    n, d = x.shape
    row = edge_index[0]
    col = edge_index[1]
    w = edge_weight.astype(jnp.float32)

    e = row.shape[0]
    pad = (-e) % EDGE_PAD
    if pad:
        zi = jnp.zeros((pad,), row.dtype)
        row = jnp.concatenate([row, zi])
        col = jnp.concatenate([col, zi])
        w = jnp.concatenate([w, jnp.zeros((pad,), w.dtype)])

    row2d = row.reshape(-1, CH)
    col2d = col.reshape(-1, CH)
    w2d = w.reshape(-1, CH)

    degp = _sc_deg_partial(col, w, n, DEG_D)
    g, dis = _tc_prep(x, W1, degp)
    agg = _sc_agg(g, row2d, col2d, w2d, n)
    g = _tc_mid(agg, g, dis, b1.reshape(1, d), W2)
    agg = _sc_agg(g, row2d, col2d, w2d, n)
    g = _tc_mid(agg, g, dis, b2.reshape(1, d), W3)
    agg = _sc_agg(g, row2d, col2d, w2d, n)
    g = _tc_mid(agg, g, dis, b3.reshape(1, d), W4)
    agg = _sc_agg(g, row2d, col2d, w2d, n)
    return _tc_final(agg, g, dis, b4.reshape(1, d))
